# baseline (speedup 1.0000x reference)
import functools

import jax
import jax.numpy as jnp
from jax import lax
from jax.experimental import pallas as pl
from jax.experimental.pallas import tpu as pltpu
from jax.experimental.pallas import tpu_sc as plsc

D_MODEL = 2048
N_EXP = 16
BT = 512
N_TOKENS = 8192
NSTEP = N_TOKENS // BT
NWORK = 32
TPW = N_TOKENS // NWORK


def _tc_body(x_ref, we_ref, be_ref, wg_ref, bg_ref, out_ref, gl_ref):
    xb = x_ref[...]
    gl = lax.dot_general(xb, wg_ref[...], (((1,), (1,)), ((), ())),
                         preferred_element_type=jnp.float32)
    gl_ref[...] = (gl + bg_ref[...]).reshape(1, BT, N_EXP)
    acc = lax.dot_general(xb, we_ref[...], (((1,), (1,)), ((), ())),
                          preferred_element_type=jnp.float32)
    out_ref[...] = acc + be_ref[...]


def _sc_route(gl_hbm, idx_hbm, w_hbm, chunk, idxc, wc):
    wid = lax.axis_index("s") * 2 + lax.axis_index("c")
    base = wid * TPW
    pltpu.sync_copy(gl_hbm.at[pl.ds(base * N_EXP, TPW * N_EXP)], chunk)
    iota16 = lax.iota(jnp.int32, 16)

    for g in range(TPW // 16):
        tok = iota16 + g * 16
        flat = tok * N_EXP
        m0v = plsc.load_gather(chunk, [flat])
        i0v = jnp.zeros((16,), jnp.int32)
        m1v = jnp.full((16,), -jnp.inf, jnp.float32)
        i1v = jnp.full((16,), N_EXP, jnp.int32)
        for e in range(1, N_EXP):
            ev_ = jnp.full((16,), e, jnp.int32)
            ve = plsc.load_gather(chunk, [flat + e])
            gt0 = ve > m0v
            gt1 = ve > m1v
            m1v = jnp.where(gt0, m0v, jnp.where(gt1, ve, m1v))
            i1v = jnp.where(gt0, i0v, jnp.where(gt1, ev_, i1v))
            m0v = jnp.where(gt0, ve, m0v)
            i0v = jnp.where(gt0, ev_, i0v)
        e1v = jnp.exp(m1v - m0v)
        sv = 1.0 + e1v
        tok2 = tok * 2
        plsc.store_scatter(idxc, [tok2], i0v)
        plsc.store_scatter(idxc, [tok2 + 1], i1v)
        plsc.store_scatter(wc, [tok2], 1.0 / sv)
        plsc.store_scatter(wc, [tok2 + 1], e1v / sv)

    pltpu.sync_copy(idxc, idx_hbm.at[pl.ds(base * 2, TPW * 2)])
    pltpu.sync_copy(wc, w_hbm.at[pl.ds(base * 2, TPW * 2)])


def kernel(x, W_exp, b_exp, W_gate, b_gate):
    n_tok = x.shape[0]
    bg2 = b_gate.reshape(1, N_EXP)
    be2 = b_exp.reshape(1, D_MODEL)

    out, glp = pl.pallas_call(
        _tc_body,
        grid=(NSTEP,),
        in_specs=[
            pl.BlockSpec((BT, D_MODEL), lambda i: (i, 0)),
            pl.BlockSpec((D_MODEL, D_MODEL), lambda i: (0, 0)),
            pl.BlockSpec((1, D_MODEL), lambda i: (0, 0)),
            pl.BlockSpec((N_EXP, D_MODEL), lambda i: (0, 0)),
            pl.BlockSpec((1, N_EXP), lambda i: (0, 0)),
        ],
        out_specs=[
            pl.BlockSpec((BT, D_MODEL), lambda i: (i, 0)),
            pl.BlockSpec((1, BT, N_EXP), lambda i: (i, 0, 0)),
        ],
        out_shape=[
            jax.ShapeDtypeStruct((n_tok, D_MODEL), jnp.float32),
            jax.ShapeDtypeStruct((NSTEP, BT, N_EXP), jnp.float32),
        ],
    )(x, W_exp, be2, W_gate, bg2)

    gl1d = glp.reshape(n_tok * N_EXP)
    mesh = plsc.VectorSubcoreMesh(core_axis_name="c", subcore_axis_name="s")
    route = functools.partial(
        pl.kernel,
        mesh=mesh,
        compiler_params=pltpu.CompilerParams(needs_layout_passes=False),
        out_type=[
            jax.ShapeDtypeStruct((n_tok * 2,), jnp.int32),
            jax.ShapeDtypeStruct((n_tok * 2,), jnp.float32),
        ],
        scratch_types=[
            pltpu.VMEM((TPW * N_EXP,), jnp.float32),
            pltpu.VMEM((TPW * 2,), jnp.int32),
            pltpu.VMEM((TPW * 2,), jnp.float32),
        ],
    )(_sc_route)
    idx, w = route(gl1d)
    return out, idx.reshape(n_tok, 2), w.reshape(n_tok, 2)


# speedup vs baseline: 1.3190x; 1.3190x over previous
import jax
import jax.numpy as jnp
from jax import lax
from jax.experimental import pallas as pl
from jax.experimental.pallas import tpu as pltpu

D_MODEL = 2048
N_EXP = 16
BT = 512


def _fused_body(x_ref, we_ref, be_ref, wg_ref, bg_ref, out_ref, idx_ref, w_ref):
    xb = x_ref[...]
    gl = lax.dot_general(xb, wg_ref[...], (((1,), (1,)), ((), ())),
                         preferred_element_type=jnp.float32)
    gl = gl + bg_ref[...]
    iota = lax.broadcasted_iota(jnp.int32, (BT, N_EXP), 1)
    m0 = jnp.max(gl, axis=1, keepdims=True)
    i0 = jnp.min(jnp.where(gl == m0, iota, N_EXP), axis=1, keepdims=True)
    gl2 = jnp.where(iota == i0, -jnp.inf, gl)
    m1 = jnp.max(gl2, axis=1, keepdims=True)
    i1 = jnp.min(jnp.where(gl2 == m1, iota, N_EXP), axis=1, keepdims=True)
    e1 = jnp.exp(m1 - m0)
    s = 1.0 + e1
    w0 = 1.0 / s
    w1 = e1 / s
    iota2 = lax.broadcasted_iota(jnp.int32, (BT, 2), 1)
    idx_ref[...] = jnp.where(iota2 == 0, i0, i1).reshape(1, BT, 2)
    w_ref[...] = jnp.where(iota2 == 0, w0, w1).reshape(1, BT, 2)
    acc = lax.dot_general(xb, we_ref[...], (((1,), (1,)), ((), ())),
                          preferred_element_type=jnp.float32)
    out_ref[...] = acc + be_ref[...]


def kernel(x, W_exp, b_exp, W_gate, b_gate):
    n_tok = x.shape[0]
    bg2 = b_gate.reshape(1, N_EXP)
    be2 = b_exp.reshape(1, D_MODEL)

    grid = (n_tok // BT,)
    out, idxp, wp = pl.pallas_call(
        _fused_body,
        grid=grid,
        in_specs=[
            pl.BlockSpec((BT, D_MODEL), lambda i: (i, 0)),
            pl.BlockSpec((D_MODEL, D_MODEL), lambda i: (0, 0)),
            pl.BlockSpec((1, D_MODEL), lambda i: (0, 0)),
            pl.BlockSpec((N_EXP, D_MODEL), lambda i: (0, 0)),
            pl.BlockSpec((1, N_EXP), lambda i: (0, 0)),
        ],
        out_specs=[
            pl.BlockSpec((BT, D_MODEL), lambda i: (i, 0)),
            pl.BlockSpec((1, BT, 2), lambda i: (i, 0, 0)),
            pl.BlockSpec((1, BT, 2), lambda i: (i, 0, 0)),
        ],
        out_shape=[
            jax.ShapeDtypeStruct((n_tok, D_MODEL), jnp.float32),
            jax.ShapeDtypeStruct((n_tok // BT, BT, 2), jnp.int32),
            jax.ShapeDtypeStruct((n_tok // BT, BT, 2), jnp.float32),
        ],
        compiler_params=pltpu.CompilerParams(
            dimension_semantics=("parallel",)),
    )(x, W_exp, be2, W_gate, bg2)
    return out, idxp.reshape(n_tok, 2), wp.reshape(n_tok, 2)
